# baseline (speedup 1.0000x reference)
import functools

import jax
import jax.numpy as jnp
from jax import lax
from jax.experimental import pallas as pl
from jax.experimental.pallas import tpu as pltpu
from jax.experimental.pallas import tpu_sc as plsc

BATCH = 16384
HIST = 50
EMBED_DIM = 64
NUM_CORES = 2
NUM_SUBCORES = 16
NUM_WORKERS = NUM_CORES * NUM_SUBCORES
ROWS_PER_WORKER = BATCH // NUM_WORKERS
GROUP = 8
NUM_GROUPS = ROWS_PER_WORKER // GROUP
NUM_PAIRS = NUM_GROUPS // 2


def _gather_kernel(x_hbm, table_hbm, out_hbm, idx_v, rows_a, rows_b,
                   gsem_a, gsem_b, wsem_a, wsem_b):
    wid = lax.axis_index("s") * NUM_CORES + lax.axis_index("c")
    r0 = wid * ROWS_PER_WORKER

    pltpu.sync_copy(x_hbm.at[pl.ds(r0, ROWS_PER_WORKER)], idx_v)

    def row_gather(g, k, rows, sem):
        return pltpu.make_async_copy(
            table_hbm.at[idx_v.at[g * GROUP + k]], rows.at[k], sem)

    def start_gathers(g, rows, sem):
        for k in range(GROUP):
            row_gather(g, k, rows, sem).start()

    def wait_gathers(g, rows, sem):
        for k in range(GROUP):
            row_gather(g, k, rows, sem).wait()

    def w_copy(g, rows, sem):
        return pltpu.make_async_copy(
            rows, out_hbm.at[pl.ds(r0 + g * GROUP, GROUP)], sem)

    start_gathers(0, rows_a, gsem_a)
    start_gathers(1, rows_b, gsem_b)

    def body(j, carry):
        g0 = 2 * j
        g1 = g0 + 1
        wait_gathers(g0, rows_a, gsem_a)
        w_copy(g0, rows_a, wsem_a).start()
        wait_gathers(g1, rows_b, gsem_b)
        w_copy(g1, rows_b, wsem_b).start()
        w_copy(g0, rows_a, wsem_a).wait()

        @pl.when(j + 1 < NUM_PAIRS)
        def _():
            start_gathers(g0 + 2, rows_a, gsem_a)

        w_copy(g1, rows_b, wsem_b).wait()

        @pl.when(j + 1 < NUM_PAIRS)
        def _():
            start_gathers(g1 + 2, rows_b, gsem_b)

        return carry

    lax.fori_loop(0, NUM_PAIRS, body, 0)


def kernel(x, table):
    idx = jnp.asarray(x, jnp.int32)
    mesh = plsc.VectorSubcoreMesh(core_axis_name="c", subcore_axis_name="s")
    run = functools.partial(
        pl.kernel,
        mesh=mesh,
        compiler_params=pltpu.CompilerParams(use_tc_tiling_on_sc=False),
        out_type=jax.ShapeDtypeStruct((BATCH, HIST, EMBED_DIM), jnp.float32),
        scratch_types=[
            pltpu.VMEM((ROWS_PER_WORKER, HIST), jnp.int32),
            pltpu.VMEM((GROUP, HIST, EMBED_DIM), jnp.float32),
            pltpu.VMEM((GROUP, HIST, EMBED_DIM), jnp.float32),
            pltpu.SemaphoreType.DMA,
            pltpu.SemaphoreType.DMA,
            pltpu.SemaphoreType.DMA,
            pltpu.SemaphoreType.DMA,
        ],
    )(_gather_kernel)
    return run(idx, table)
